# baseline (speedup 1.0000x reference)
import functools

import jax
import jax.numpy as jnp
from jax import lax
from jax.experimental import pallas as pl
from jax.experimental.pallas import tpu as pltpu
from jax.experimental.pallas import tpu_sc as plsc

_B, _C, _N, _K, _O = 8, 64, 2048, 16, 64
_BN = _B * _N
_RB = 256
_NEG = float("-inf")

_NC, _NS = 2, 16
_NW = _NC * _NS
_PPW = _BN // _NW
_G = 32
_NGATHER = _G * _K // 128
_NCH = _PPW // _G

_SB = 256
_NSB = _BN // _SB


def _knn_body(xfull_ref, xrows_ref, w_ref, idx_ref, z12_ref):
    b = pl.program_id(0)
    rb = pl.program_id(1)
    xfull = xfull_ref[0]
    xrows = xrows_ref[0]
    w = w_ref[...]
    w1 = w[:, :_C]
    w2 = w[:, _C:]

    zz1 = lax.dot_general(xrows, w1, (((0,), (1,)), ((), ())),
                          preferred_element_type=jnp.float32)
    zz2 = lax.dot_general(xrows, w2 - w1, (((0,), (1,)), ((), ())),
                          preferred_element_type=jnp.float32)
    z12_ref[0] = jnp.concatenate([zz1, zz2], axis=1)

    inner = lax.dot_general(xrows, xfull, (((0,), (0,)), ((), ())),
                            preferred_element_type=jnp.float32)
    sq_full = jnp.sum(xfull * xfull, axis=0, keepdims=True)
    sq_rows = jnp.sum(xrows * xrows, axis=0, keepdims=True)
    dist = (-sq_rows.T - sq_full) + 2.0 * inner

    col = lax.broadcasted_iota(jnp.int32, (_RB, _N), 1)
    row_g = rb * _RB + lax.broadcasted_iota(jnp.int32, (_RB, _N), 0)
    vals = jnp.where(col == row_g, _NEG, dist)

    colf = col.astype(jnp.float32)
    lane = lax.broadcasted_iota(jnp.int32, (_RB, _K), 1)
    idsf = jnp.zeros((_RB, _K), jnp.float32)
    for t in range(_K):
        m = jnp.max(vals, axis=1, keepdims=True)
        cand = jnp.where(vals == m, colf, jnp.float32(_N))
        a = jnp.min(cand, axis=1, keepdims=True)
        idsf = jnp.where(lane == t, a, idsf)
        vals = jnp.where(colf == a, _NEG, vals)
    idx_ref[0] = idsf.astype(jnp.int32) + b * _N


def _knn_call(x, w):
    return pl.pallas_call(
        _knn_body,
        grid=(_B, _N // _RB),
        in_specs=[
            pl.BlockSpec((1, _C, _N), lambda b, r: (b, 0, 0)),
            pl.BlockSpec((1, _C, _RB), lambda b, r: (b, 0, r)),
            pl.BlockSpec((_O, 2 * _C), lambda b, r: (0, 0)),
        ],
        out_specs=[
            pl.BlockSpec((1, _RB, _K), lambda b, r: (b, r, 0)),
            pl.BlockSpec((1, _RB, 2 * _O), lambda b, r: (b, r, 0)),
        ],
        out_shape=[
            jax.ShapeDtypeStruct((_B, _N, _K), jnp.int32),
            jax.ShapeDtypeStruct((_B, _N, 2 * _O), jnp.float32),
        ],
    )(x, x, w)


def _sc_body(z12_hbm, idx_hbm, g_hbm, idx_v, rows_v, sem):
    cid = lax.axis_index("c")
    sid = lax.axis_index("s")
    wid = sid * _NC + cid
    base = wid * _PPW

    @pl.loop(0, _NCH)
    def _chunk(ci):
        p0 = base + ci * _G
        for q in range(_NGATHER):
            pltpu.sync_copy(
                idx_hbm.at[pl.ds(p0 * _K + q * 128, 128)], idx_v.at[q])
        descs = [
            pltpu.async_copy(z12_hbm.at[idx_v.at[q]],
                             rows_v.at[pl.ds(q * 128, 128)], sem)
            for q in range(_NGATHER)
        ]
        for d in descs:
            d.wait()
        pltpu.sync_copy(rows_v, g_hbm.at[pl.ds(p0 * _K, _G * _K)])


def _sc_call(z12f, idxf):
    mesh = plsc.VectorSubcoreMesh(
        core_axis_name="c", subcore_axis_name="s",
        num_cores=_NC, num_subcores=_NS)
    f = pl.kernel(
        _sc_body,
        out_type=[
            jax.ShapeDtypeStruct((_BN * _K, 2 * _O), jnp.float32),
        ],
        mesh=mesh,
        scratch_types=[
            pltpu.VMEM((_NGATHER, 128), jnp.int32),
            pltpu.VMEM((_G * _K, 2 * _O), jnp.float32),
            pltpu.SemaphoreType.DMA,
        ],
    )
    return f(z12f, idxf)


def _stats_body(g_ref, z12_ref, ymax_ref, ymin_ref, sums_ref):
    g = g_ref[:, :_O].reshape(_SB, _K, _O)
    z2 = z12_ref[:, _O:]
    mx = jnp.max(g, axis=1)
    mn = jnp.min(g, axis=1)
    sm = jnp.sum(g, axis=1)
    sq = jnp.sum(g * g, axis=1)
    ymax_ref[...] = mx + z2
    ymin_ref[...] = mn + z2
    kf = jnp.float32(_K)
    s_a = sm + kf * z2
    s_b = sq + 2.0 * z2 * sm + kf * z2 * z2
    sval = jnp.concatenate(
        [jnp.sum(s_a, axis=0, keepdims=True),
         jnp.sum(s_b, axis=0, keepdims=True)], axis=1)
    row = lax.broadcasted_iota(jnp.int32, (8, 2 * _O), 0)
    sums_ref[0] = jnp.where(row == 0, sval, 0.0)


def _stats_call(g, z12f):
    return pl.pallas_call(
        _stats_body,
        grid=(_NSB,),
        in_specs=[
            pl.BlockSpec((_SB * _K, 2 * _O), lambda i: (i, 0)),
            pl.BlockSpec((_SB, 2 * _O), lambda i: (i, 0)),
        ],
        out_specs=[
            pl.BlockSpec((_SB, _O), lambda i: (i, 0)),
            pl.BlockSpec((_SB, _O), lambda i: (i, 0)),
            pl.BlockSpec((1, 8, 2 * _O), lambda i: (i, 0, 0)),
        ],
        out_shape=[
            jax.ShapeDtypeStruct((_BN, _O), jnp.float32),
            jax.ShapeDtypeStruct((_BN, _O), jnp.float32),
            jax.ShapeDtypeStruct((_NSB, 8, 2 * _O), jnp.float32),
        ],
    )(g, z12f)


def _final_body(ymax_ref, ymin_ref, sums_ref, gamma_ref, beta_ref, out_ref):
    sums = sums_ref[...].reshape(_NSB * 8, 2 * _O)
    cnt = jnp.float32(_BN * _K)
    s_a = jnp.sum(sums[:, :_O], axis=0, keepdims=True)
    s_b = jnp.sum(sums[:, _O:], axis=0, keepdims=True)
    mean = s_a / cnt
    var = s_b / cnt - mean * mean
    inv = 1.0 / jnp.sqrt(var + 1e-5)
    slope = gamma_ref[...] * inv
    intercept = beta_ref[...] - mean * slope

    ymax = ymax_ref[0]
    ymin = ymin_ref[0]
    ext = jnp.where(slope >= 0.0, ymax, ymin)
    yn = ext * slope + intercept
    act = jnp.where(yn >= 0.0, yn, 0.2 * yn)

    eye = (lax.broadcasted_iota(jnp.int32, (_O, _O), 0)
           == lax.broadcasted_iota(jnp.int32, (_O, _O), 1)
           ).astype(jnp.float32)
    out_ref[0] = lax.dot_general(eye, act, (((1,), (1,)), ((), ())),
                                 preferred_element_type=jnp.float32)


def _final_call(ymax, ymin, sums, gamma, beta):
    return pl.pallas_call(
        _final_body,
        grid=(_B,),
        in_specs=[
            pl.BlockSpec((1, _N, _O), lambda b: (b, 0, 0)),
            pl.BlockSpec((1, _N, _O), lambda b: (b, 0, 0)),
            pl.BlockSpec((_NSB, 8, 2 * _O), lambda b: (0, 0, 0)),
            pl.BlockSpec((1, _O), lambda b: (0, 0)),
            pl.BlockSpec((1, _O), lambda b: (0, 0)),
        ],
        out_specs=pl.BlockSpec((1, _O, _N), lambda b: (b, 0, 0)),
        out_shape=jax.ShapeDtypeStruct((_B, _O, _N), jnp.float32),
    )(ymax, ymin, sums, gamma, beta)


def kernel(x, W, gamma, beta):
    idx, z12 = _knn_call(x, W)
    z12f = z12.reshape(_BN, 2 * _O)
    (g,) = _sc_call(z12f, idx.reshape(_BN * _K))
    ymax, ymin, sums = _stats_call(g, z12f)
    return _final_call(
        ymax.reshape(_B, _N, _O), ymin.reshape(_B, _N, _O),
        sums, gamma.reshape(1, _O), beta.reshape(1, _O))


# speedup vs baseline: 1.1934x; 1.1934x over previous
import functools

import jax
import jax.numpy as jnp
from jax import lax
from jax.experimental import pallas as pl
from jax.experimental.pallas import tpu as pltpu
from jax.experimental.pallas import tpu_sc as plsc

_B, _C, _N, _K, _O = 8, 64, 2048, 16, 64
_BN = _B * _N
_RB = 256
_NEG = float("-inf")

_NC, _NS = 2, 16
_NW = _NC * _NS
_PPW = _BN // _NW
_G = 32
_NGATHER = _G * _K // 128
_NCH = _PPW // _G


def _knn_body(xfull_ref, xrows_ref, w_ref, idx_ref, z12_ref):
    b = pl.program_id(0)
    rb = pl.program_id(1)
    xfull = xfull_ref[0]
    xrows = xrows_ref[0]
    w = w_ref[...]
    w1 = w[:, :_C]
    w2 = w[:, _C:]

    zz1 = lax.dot_general(xrows, w1, (((0,), (1,)), ((), ())),
                          preferred_element_type=jnp.float32)
    zz2 = lax.dot_general(xrows, w2 - w1, (((0,), (1,)), ((), ())),
                          preferred_element_type=jnp.float32)
    z12_ref[0] = jnp.concatenate([zz1, zz2], axis=1)

    inner = lax.dot_general(xrows, xfull, (((0,), (0,)), ((), ())),
                            preferred_element_type=jnp.float32)
    sq_full = jnp.sum(xfull * xfull, axis=0, keepdims=True)
    sq_rows = jnp.sum(xrows * xrows, axis=0, keepdims=True)
    dist = (-sq_rows.T - sq_full) + 2.0 * inner

    col = lax.broadcasted_iota(jnp.int32, (_RB, _N), 1)
    row_g = rb * _RB + lax.broadcasted_iota(jnp.int32, (_RB, _N), 0)
    vals = jnp.where(col == row_g, _NEG, dist)

    colf = col.astype(jnp.float32)
    lane = lax.broadcasted_iota(jnp.int32, (_RB, _K), 1)
    idsf = jnp.zeros((_RB, _K), jnp.float32)
    for t in range(_K):
        m = jnp.max(vals, axis=1, keepdims=True)
        cand = jnp.where(vals == m, colf, jnp.float32(_N))
        a = jnp.min(cand, axis=1, keepdims=True)
        idsf = jnp.where(lane == t, a, idsf)
        vals = jnp.where(colf == a, _NEG, vals)
    idx_ref[0] = idsf.astype(jnp.int32) + b * _N


def _knn_call(x, w):
    return pl.pallas_call(
        _knn_body,
        grid=(_B, _N // _RB),
        in_specs=[
            pl.BlockSpec((1, _C, _N), lambda b, r: (b, 0, 0)),
            pl.BlockSpec((1, _C, _RB), lambda b, r: (b, 0, r)),
            pl.BlockSpec((_O, 2 * _C), lambda b, r: (0, 0)),
        ],
        out_specs=[
            pl.BlockSpec((1, _RB, _K), lambda b, r: (b, r, 0)),
            pl.BlockSpec((1, _RB, 2 * _O), lambda b, r: (b, r, 0)),
        ],
        out_shape=[
            jax.ShapeDtypeStruct((_B, _N, _K), jnp.int32),
            jax.ShapeDtypeStruct((_B, _N, 2 * _O), jnp.float32),
        ],
    )(x, x, w)


def _sc_body(z12_hbm, idx_hbm, ymax_hbm, ymin_hbm, part_hbm,
             idx_v, rows_v, cent_v, ymax_v, ymin_v, sums_v, out_v, sem):
    cid = lax.axis_index("c")
    sid = lax.axis_index("s")
    wid = sid * _NC + cid
    base = wid * _PPW

    zeros = jnp.zeros((16,), jnp.float32)
    for r in range(8):
        sums_v[r] = zeros

    @pl.loop(0, _NCH)
    def _chunk(ci):
        p0 = base + ci * _G
        for q in range(_NGATHER):
            pltpu.sync_copy(
                idx_hbm.at[pl.ds(p0 * _K + q * 128, 128)], idx_v.at[q])
        descs = [
            pltpu.async_copy(z12_hbm.at[idx_v.at[q]],
                             rows_v.at[pl.ds(q * 128, 128)], sem)
            for q in range(_NGATHER)
        ]
        pltpu.sync_copy(z12_hbm.at[pl.ds(p0, _G)], cent_v)
        for d in descs:
            d.wait()

        @pl.loop(0, _G)
        def _pt(p):
            r0 = p * _K
            for c in range(4):
                sl = pl.ds(c * 16, 16)
                v = rows_v[r0, sl]
                mx = v
                mn = v
                sm = v
                sq = v * v
                for j in range(1, _K):
                    v = rows_v[r0 + j, sl]
                    mx = jnp.maximum(mx, v)
                    mn = jnp.minimum(mn, v)
                    sm = sm + v
                    sq = sq + v * v
                z2c = cent_v[p, pl.ds(_O + c * 16, 16)]
                ymax_v[p, sl] = mx + z2c
                ymin_v[p, sl] = mn + z2c
                sums_v[c] = sums_v[c] + sm + 16.0 * z2c
                sums_v[4 + c] = (sums_v[4 + c] + sq
                                 + 2.0 * z2c * sm + 16.0 * z2c * z2c)

        pltpu.sync_copy(ymax_v, ymax_hbm.at[pl.ds(p0, _G)])
        pltpu.sync_copy(ymin_v, ymin_hbm.at[pl.ds(p0, _G)])

    for c in range(4):
        out_v[0, pl.ds(c * 16, 16)] = sums_v[c]
        out_v[0, pl.ds(_O + c * 16, 16)] = sums_v[4 + c]
    pltpu.sync_copy(out_v, part_hbm.at[pl.ds(8 * wid, 1)])


def _sc_call(z12f, idxf):
    mesh = plsc.VectorSubcoreMesh(
        core_axis_name="c", subcore_axis_name="s",
        num_cores=_NC, num_subcores=_NS)
    f = pl.kernel(
        _sc_body,
        out_type=[
            jax.ShapeDtypeStruct((_BN, _O), jnp.float32),
            jax.ShapeDtypeStruct((_BN, _O), jnp.float32),
            jax.ShapeDtypeStruct((_NW * 8, 2 * _O), jnp.float32),
        ],
        mesh=mesh,
        scratch_types=[
            pltpu.VMEM((_NGATHER, 128), jnp.int32),
            pltpu.VMEM((_G * _K, 2 * _O), jnp.float32),
            pltpu.VMEM((_G, 2 * _O), jnp.float32),
            pltpu.VMEM((_G, _O), jnp.float32),
            pltpu.VMEM((_G, _O), jnp.float32),
            pltpu.VMEM((8, 16), jnp.float32),
            pltpu.VMEM((1, 2 * _O), jnp.float32),
            pltpu.SemaphoreType.DMA,
        ],
    )
    return f(z12f, idxf)


def _final_body(ymax_ref, ymin_ref, sums_ref, gamma_ref, beta_ref, out_ref):
    sums = sums_ref[...].reshape(_NW, 8, 2 * _O)[:, 0, :]
    cnt = jnp.float32(_BN * _K)
    s_a = jnp.sum(sums[:, :_O], axis=0, keepdims=True)
    s_b = jnp.sum(sums[:, _O:], axis=0, keepdims=True)
    mean = s_a / cnt
    var = s_b / cnt - mean * mean
    inv = 1.0 / jnp.sqrt(var + 1e-5)
    slope = gamma_ref[...] * inv
    intercept = beta_ref[...] - mean * slope

    ymax = ymax_ref[0]
    ymin = ymin_ref[0]
    ext = jnp.where(slope >= 0.0, ymax, ymin)
    yn = ext * slope + intercept
    act = jnp.where(yn >= 0.0, yn, 0.2 * yn)

    eye = (lax.broadcasted_iota(jnp.int32, (_O, _O), 0)
           == lax.broadcasted_iota(jnp.int32, (_O, _O), 1)
           ).astype(jnp.float32)
    out_ref[0] = lax.dot_general(eye, act, (((1,), (1,)), ((), ())),
                                 preferred_element_type=jnp.float32)


def _final_call(ymax, ymin, sums, gamma, beta):
    return pl.pallas_call(
        _final_body,
        grid=(_B,),
        in_specs=[
            pl.BlockSpec((1, _N, _O), lambda b: (b, 0, 0)),
            pl.BlockSpec((1, _N, _O), lambda b: (b, 0, 0)),
            pl.BlockSpec((_NW * 8, 2 * _O), lambda b: (0, 0)),
            pl.BlockSpec((1, _O), lambda b: (0, 0)),
            pl.BlockSpec((1, _O), lambda b: (0, 0)),
        ],
        out_specs=pl.BlockSpec((1, _O, _N), lambda b: (b, 0, 0)),
        out_shape=jax.ShapeDtypeStruct((_B, _O, _N), jnp.float32),
    )(ymax, ymin, sums, gamma, beta)


def kernel(x, W, gamma, beta):
    idx, z12 = _knn_call(x, W)
    z12f = z12.reshape(_BN, 2 * _O)
    ymax, ymin, part = _sc_call(z12f, idx.reshape(_BN * _K))
    return _final_call(
        ymax.reshape(_B, _N, _O), ymin.reshape(_B, _N, _O),
        part, gamma.reshape(1, _O), beta.reshape(1, _O))


# speedup vs baseline: 1.3239x; 1.1094x over previous
import functools

import jax
import jax.numpy as jnp
from jax import lax
from jax.experimental import pallas as pl
from jax.experimental.pallas import tpu as pltpu
from jax.experimental.pallas import tpu_sc as plsc

_B, _C, _N, _K, _O = 8, 64, 2048, 16, 64
_BN = _B * _N
_RB = 256
_NEG = float("-inf")

_NC, _NS = 2, 16
_NW = _NC * _NS
_PPW = _BN // _NW
_G = 32
_NGATHER = _G * _K // 128
_NCH = _PPW // _G


def _knn_body(xfull_ref, xrows_ref, w_ref, idx_ref, z12_ref):
    b = pl.program_id(0)
    rb = pl.program_id(1)
    xfull = xfull_ref[0]
    xrows = xrows_ref[0]
    w = w_ref[...]
    w1 = w[:, :_C]
    w2 = w[:, _C:]

    zz1 = lax.dot_general(xrows, w1, (((0,), (1,)), ((), ())),
                          preferred_element_type=jnp.float32)
    zz2 = lax.dot_general(xrows, w2 - w1, (((0,), (1,)), ((), ())),
                          preferred_element_type=jnp.float32)
    z12_ref[0] = jnp.concatenate([zz1, zz2], axis=1)

    inner = lax.dot_general(xrows, xfull, (((0,), (0,)), ((), ())),
                            preferred_element_type=jnp.float32)
    sq_full = jnp.sum(xfull * xfull, axis=0, keepdims=True)
    sq_rows = jnp.sum(xrows * xrows, axis=0, keepdims=True)
    dist = (-sq_rows.T - sq_full) + 2.0 * inner

    col = lax.broadcasted_iota(jnp.int32, (_RB, _N), 1)
    row_g = rb * _RB + lax.broadcasted_iota(jnp.int32, (_RB, _N), 0)
    vals = jnp.where(col == row_g, _NEG, dist)

    colf = col.astype(jnp.float32)
    lane = lax.broadcasted_iota(jnp.int32, (_RB, _K), 1)
    idsf = jnp.zeros((_RB, _K), jnp.float32)
    for t in range(_K):
        m = jnp.max(vals, axis=1, keepdims=True)
        cand = jnp.where(vals == m, colf, jnp.float32(_N))
        a = jnp.min(cand, axis=1, keepdims=True)
        idsf = jnp.where(lane == t, a, idsf)
        vals = jnp.where(colf == a, _NEG, vals)
    idx_ref[0] = idsf.astype(jnp.int32) + b * _N


def _knn_call(x, w):
    nb = x.shape[0]
    return pl.pallas_call(
        _knn_body,
        grid=(nb, _N // _RB),
        in_specs=[
            pl.BlockSpec((1, _C, _N), lambda b, r: (b, 0, 0)),
            pl.BlockSpec((1, _C, _RB), lambda b, r: (b, 0, r)),
            pl.BlockSpec((_O, 2 * _C), lambda b, r: (0, 0)),
        ],
        out_specs=[
            pl.BlockSpec((1, _RB, _K), lambda b, r: (b, r, 0)),
            pl.BlockSpec((1, _RB, 2 * _O), lambda b, r: (b, r, 0)),
        ],
        out_shape=[
            jax.ShapeDtypeStruct((nb, _N, _K), jnp.int32),
            jax.ShapeDtypeStruct((nb, _N, 2 * _O), jnp.float32),
        ],
    )(x, x, w)


def _sc_body(ppw, nch, z12_hbm, idx_hbm, ymax_hbm, ymin_hbm, part_hbm,
             idx_v, rows_v, cent_v, ymax_v, ymin_v, sums_v, out_v, sem):
    cid = lax.axis_index("c")
    sid = lax.axis_index("s")
    wid = sid * _NC + cid
    base = wid * ppw

    zeros = jnp.zeros((16,), jnp.float32)
    for r in range(8):
        sums_v[r] = zeros

    @pl.loop(0, nch)
    def _chunk(ci):
        p0 = base + ci * _G
        for q in range(_NGATHER):
            pltpu.sync_copy(
                idx_hbm.at[pl.ds(p0 * _K + q * 128, 128)], idx_v.at[q])
        descs = [
            pltpu.async_copy(z12_hbm.at[idx_v.at[q]],
                             rows_v.at[pl.ds(q * 128, 128)], sem)
            for q in range(_NGATHER)
        ]
        pltpu.sync_copy(z12_hbm.at[pl.ds(p0, _G)], cent_v)
        for d in descs:
            d.wait()

        @pl.loop(0, _G)
        def _pt(p):
            r0 = p * _K
            for c in range(4):
                sl = pl.ds(c * 16, 16)
                v = rows_v[r0, sl]
                mx = v
                mn = v
                sm = v
                sq = v * v
                for j in range(1, _K):
                    v = rows_v[r0 + j, sl]
                    mx = jnp.maximum(mx, v)
                    mn = jnp.minimum(mn, v)
                    sm = sm + v
                    sq = sq + v * v
                z2c = cent_v[p, pl.ds(_O + c * 16, 16)]
                ymax_v[p, sl] = mx + z2c
                ymin_v[p, sl] = mn + z2c
                sums_v[c] = sums_v[c] + sm + 16.0 * z2c
                sums_v[4 + c] = (sums_v[4 + c] + sq
                                 + 2.0 * z2c * sm + 16.0 * z2c * z2c)

        pltpu.sync_copy(ymax_v, ymax_hbm.at[pl.ds(p0, _G)])
        pltpu.sync_copy(ymin_v, ymin_hbm.at[pl.ds(p0, _G)])

    for c in range(4):
        out_v[0, pl.ds(c * 16, 16)] = sums_v[c]
        out_v[0, pl.ds(_O + c * 16, 16)] = sums_v[4 + c]
    pltpu.sync_copy(out_v, part_hbm.at[pl.ds(8 * wid, 1)])


def _sc_call(z12f, idxf):
    bn = z12f.shape[0]
    ppw = bn // _NW
    nch = ppw // _G
    mesh = plsc.VectorSubcoreMesh(
        core_axis_name="c", subcore_axis_name="s",
        num_cores=_NC, num_subcores=_NS)
    f = pl.kernel(
        functools.partial(_sc_body, ppw, nch),
        out_type=[
            jax.ShapeDtypeStruct((bn, _O), jnp.float32),
            jax.ShapeDtypeStruct((bn, _O), jnp.float32),
            jax.ShapeDtypeStruct((_NW * 8, 2 * _O), jnp.float32),
        ],
        mesh=mesh,
        scratch_types=[
            pltpu.VMEM((_NGATHER, 128), jnp.int32),
            pltpu.VMEM((_G * _K, 2 * _O), jnp.float32),
            pltpu.VMEM((_G, 2 * _O), jnp.float32),
            pltpu.VMEM((_G, _O), jnp.float32),
            pltpu.VMEM((_G, _O), jnp.float32),
            pltpu.VMEM((8, 16), jnp.float32),
            pltpu.VMEM((1, 2 * _O), jnp.float32),
            pltpu.SemaphoreType.DMA,
        ],
    )
    return f(z12f, idxf)


_HB = _B // 2


def _final_body(ya_ref, yb_ref, ia_ref, ib_ref, pa_ref, pb_ref,
                gamma_ref, beta_ref, out_ref):
    pa = pa_ref[...].reshape(_NW, 8, 2 * _O)[:, 0, :]
    pb = pb_ref[...].reshape(_NW, 8, 2 * _O)[:, 0, :]
    cnt = jnp.float32(_BN * _K)
    s_a = (jnp.sum(pa[:, :_O], axis=0, keepdims=True)
           + jnp.sum(pb[:, :_O], axis=0, keepdims=True))
    s_b = (jnp.sum(pa[:, _O:], axis=0, keepdims=True)
           + jnp.sum(pb[:, _O:], axis=0, keepdims=True))
    mean = s_a / cnt
    var = s_b / cnt - mean * mean
    inv = 1.0 / jnp.sqrt(var + 1e-5)
    slope = gamma_ref[...] * inv
    intercept = beta_ref[...] - mean * slope

    first = pl.program_id(0) < _HB
    ymax = jnp.where(first, ya_ref[0], yb_ref[0])
    ymin = jnp.where(first, ia_ref[0], ib_ref[0])
    ext = jnp.where(slope >= 0.0, ymax, ymin)
    yn = ext * slope + intercept
    act = jnp.where(yn >= 0.0, yn, 0.2 * yn)

    eye = (lax.broadcasted_iota(jnp.int32, (_O, _O), 0)
           == lax.broadcasted_iota(jnp.int32, (_O, _O), 1)
           ).astype(jnp.float32)
    out_ref[0] = lax.dot_general(eye, act, (((1,), (1,)), ((), ())),
                                 preferred_element_type=jnp.float32)


def _final_call(ymax_a, ymax_b, ymin_a, ymin_b, part_a, part_b, gamma, beta):
    half_a = lambda b: (jnp.minimum(b, _HB - 1), 0, 0)
    half_b = lambda b: (jnp.maximum(b - _HB, 0), 0, 0)
    return pl.pallas_call(
        _final_body,
        grid=(_B,),
        in_specs=[
            pl.BlockSpec((1, _N, _O), half_a),
            pl.BlockSpec((1, _N, _O), half_b),
            pl.BlockSpec((1, _N, _O), half_a),
            pl.BlockSpec((1, _N, _O), half_b),
            pl.BlockSpec((_NW * 8, 2 * _O), lambda b: (0, 0)),
            pl.BlockSpec((_NW * 8, 2 * _O), lambda b: (0, 0)),
            pl.BlockSpec((1, _O), lambda b: (0, 0)),
            pl.BlockSpec((1, _O), lambda b: (0, 0)),
        ],
        out_specs=pl.BlockSpec((1, _O, _N), lambda b: (b, 0, 0)),
        out_shape=jax.ShapeDtypeStruct((_B, _O, _N), jnp.float32),
    )(ymax_a, ymax_b, ymin_a, ymin_b, part_a, part_b, gamma, beta)


def kernel(x, W, gamma, beta):
    bnh = _HB * _N
    idx_a, z12_a = _knn_call(x[:_HB], W)
    ymax_a, ymin_a, part_a = _sc_call(
        z12_a.reshape(bnh, 2 * _O), idx_a.reshape(bnh * _K))
    idx_b, z12_b = _knn_call(x[_HB:], W)
    ymax_b, ymin_b, part_b = _sc_call(
        z12_b.reshape(bnh, 2 * _O), idx_b.reshape(bnh * _K))
    return _final_call(
        ymax_a.reshape(_HB, _N, _O), ymax_b.reshape(_HB, _N, _O),
        ymin_a.reshape(_HB, _N, _O), ymin_b.reshape(_HB, _N, _O),
        part_a, part_b, gamma.reshape(1, _O), beta.reshape(1, _O))


# speedup vs baseline: 1.3253x; 1.0011x over previous
import functools

import jax
import jax.numpy as jnp
from jax import lax
from jax.experimental import pallas as pl
from jax.experimental.pallas import tpu as pltpu
from jax.experimental.pallas import tpu_sc as plsc

_B, _C, _N, _K, _O = 8, 64, 2048, 16, 64
_BN = _B * _N
_RB = 256
_NEG = float("-inf")

_NC, _NS = 2, 16
_NW = _NC * _NS
_PPW = _BN // _NW
_G = 32
_NGATHER = _G * _K // 128
_NCH = _PPW // _G


def _knn_body(xfull_ref, xrows_ref, w_ref, idx_ref, z12_ref):
    b = pl.program_id(0)
    rb = pl.program_id(1)
    xfull = xfull_ref[0]
    xrows = xrows_ref[0]
    w = w_ref[...]
    w1 = w[:, :_C]
    w2 = w[:, _C:]

    zz1 = lax.dot_general(xrows, w1, (((0,), (1,)), ((), ())),
                          preferred_element_type=jnp.float32)
    zz2 = lax.dot_general(xrows, w2 - w1, (((0,), (1,)), ((), ())),
                          preferred_element_type=jnp.float32)
    z12_ref[0] = jnp.concatenate([zz1, zz2], axis=1)

    inner = lax.dot_general(xrows, xfull, (((0,), (0,)), ((), ())),
                            preferred_element_type=jnp.float32)
    sq_full = jnp.sum(xfull * xfull, axis=0, keepdims=True)
    sq_rows = jnp.sum(xrows * xrows, axis=0, keepdims=True)
    dist = (-sq_rows.T - sq_full) + 2.0 * inner

    col = lax.broadcasted_iota(jnp.int32, (_RB, _N), 1)
    row_g = rb * _RB + lax.broadcasted_iota(jnp.int32, (_RB, _N), 0)
    vals = jnp.where(col == row_g, _NEG, dist)

    colf = col.astype(jnp.float32)
    lane = lax.broadcasted_iota(jnp.int32, (_RB, _K), 1)
    idsf = jnp.zeros((_RB, _K), jnp.float32)
    for t in range(_K):
        m = jnp.max(vals, axis=1, keepdims=True)
        cand = jnp.where(vals == m, colf, jnp.float32(_N))
        a = jnp.min(cand, axis=1, keepdims=True)
        idsf = jnp.where(lane == t, a, idsf)
        vals = jnp.where(colf == a, _NEG, vals)
    idx_ref[0] = idsf.astype(jnp.int32) + b * _N


def _knn_call(x, w):
    nb = x.shape[0]
    return pl.pallas_call(
        _knn_body,
        grid=(nb, _N // _RB),
        compiler_params=pltpu.CompilerParams(
            dimension_semantics=("parallel", "parallel")),
        in_specs=[
            pl.BlockSpec((1, _C, _N), lambda b, r: (b, 0, 0)),
            pl.BlockSpec((1, _C, _RB), lambda b, r: (b, 0, r)),
            pl.BlockSpec((_O, 2 * _C), lambda b, r: (0, 0)),
        ],
        out_specs=[
            pl.BlockSpec((1, _RB, _K), lambda b, r: (b, r, 0)),
            pl.BlockSpec((1, _RB, 2 * _O), lambda b, r: (b, r, 0)),
        ],
        out_shape=[
            jax.ShapeDtypeStruct((nb, _N, _K), jnp.int32),
            jax.ShapeDtypeStruct((nb, _N, 2 * _O), jnp.float32),
        ],
    )(x, x, w)


def _sc_body(ppw, nch, z12_hbm, idx_hbm, ymax_hbm, ymin_hbm, part_hbm,
             idx_v, rows_v, cent_v, ymax_v, ymin_v, sums_v, out_v, sem):
    cid = lax.axis_index("c")
    sid = lax.axis_index("s")
    wid = sid * _NC + cid
    base = wid * ppw

    zeros = jnp.zeros((16,), jnp.float32)
    for r in range(8):
        sums_v[r] = zeros

    @pl.loop(0, nch)
    def _chunk(ci):
        p0 = base + ci * _G
        for q in range(_NGATHER):
            pltpu.sync_copy(
                idx_hbm.at[pl.ds(p0 * _K + q * 128, 128)], idx_v.at[q])
        descs = [
            pltpu.async_copy(z12_hbm.at[idx_v.at[q]],
                             rows_v.at[pl.ds(q * 128, 128)], sem)
            for q in range(_NGATHER)
        ]
        pltpu.sync_copy(z12_hbm.at[pl.ds(p0, _G)], cent_v)
        for d in descs:
            d.wait()

        @pl.loop(0, _G)
        def _pt(p):
            r0 = p * _K
            for c in range(4):
                sl = pl.ds(c * 16, 16)
                v = rows_v[r0, sl]
                mx = v
                mn = v
                sm = v
                sq = v * v
                for j in range(1, _K):
                    v = rows_v[r0 + j, sl]
                    mx = jnp.maximum(mx, v)
                    mn = jnp.minimum(mn, v)
                    sm = sm + v
                    sq = sq + v * v
                z2c = cent_v[p, pl.ds(_O + c * 16, 16)]
                ymax_v[p, sl] = mx + z2c
                ymin_v[p, sl] = mn + z2c
                sums_v[c] = sums_v[c] + sm + 16.0 * z2c
                sums_v[4 + c] = (sums_v[4 + c] + sq
                                 + 2.0 * z2c * sm + 16.0 * z2c * z2c)

        pltpu.sync_copy(ymax_v, ymax_hbm.at[pl.ds(p0, _G)])
        pltpu.sync_copy(ymin_v, ymin_hbm.at[pl.ds(p0, _G)])

    for c in range(4):
        out_v[0, pl.ds(c * 16, 16)] = sums_v[c]
        out_v[0, pl.ds(_O + c * 16, 16)] = sums_v[4 + c]
    pltpu.sync_copy(out_v, part_hbm.at[pl.ds(8 * wid, 1)])


def _sc_call(z12f, idxf):
    bn = z12f.shape[0]
    ppw = bn // _NW
    nch = ppw // _G
    mesh = plsc.VectorSubcoreMesh(
        core_axis_name="c", subcore_axis_name="s",
        num_cores=_NC, num_subcores=_NS)
    f = pl.kernel(
        functools.partial(_sc_body, ppw, nch),
        out_type=[
            jax.ShapeDtypeStruct((bn, _O), jnp.float32),
            jax.ShapeDtypeStruct((bn, _O), jnp.float32),
            jax.ShapeDtypeStruct((_NW * 8, 2 * _O), jnp.float32),
        ],
        mesh=mesh,
        scratch_types=[
            pltpu.VMEM((_NGATHER, 128), jnp.int32),
            pltpu.VMEM((_G * _K, 2 * _O), jnp.float32),
            pltpu.VMEM((_G, 2 * _O), jnp.float32),
            pltpu.VMEM((_G, _O), jnp.float32),
            pltpu.VMEM((_G, _O), jnp.float32),
            pltpu.VMEM((8, 16), jnp.float32),
            pltpu.VMEM((1, 2 * _O), jnp.float32),
            pltpu.SemaphoreType.DMA,
        ],
    )
    return f(z12f, idxf)


_HB = _B // 2


def _final_body(ya_ref, yb_ref, ia_ref, ib_ref, pa_ref, pb_ref,
                gamma_ref, beta_ref, out_ref):
    pa = pa_ref[...].reshape(_NW, 8, 2 * _O)[:, 0, :]
    pb = pb_ref[...].reshape(_NW, 8, 2 * _O)[:, 0, :]
    cnt = jnp.float32(_BN * _K)
    s_a = (jnp.sum(pa[:, :_O], axis=0, keepdims=True)
           + jnp.sum(pb[:, :_O], axis=0, keepdims=True))
    s_b = (jnp.sum(pa[:, _O:], axis=0, keepdims=True)
           + jnp.sum(pb[:, _O:], axis=0, keepdims=True))
    mean = s_a / cnt
    var = s_b / cnt - mean * mean
    inv = 1.0 / jnp.sqrt(var + 1e-5)
    slope = gamma_ref[...] * inv
    intercept = beta_ref[...] - mean * slope

    first = pl.program_id(0) < _HB
    ymax = jnp.where(first, ya_ref[0], yb_ref[0])
    ymin = jnp.where(first, ia_ref[0], ib_ref[0])
    ext = jnp.where(slope >= 0.0, ymax, ymin)
    yn = ext * slope + intercept
    act = jnp.where(yn >= 0.0, yn, 0.2 * yn)

    eye = (lax.broadcasted_iota(jnp.int32, (_O, _O), 0)
           == lax.broadcasted_iota(jnp.int32, (_O, _O), 1)
           ).astype(jnp.float32)
    out_ref[0] = lax.dot_general(eye, act, (((1,), (1,)), ((), ())),
                                 preferred_element_type=jnp.float32)


def _final_call(ymax_a, ymax_b, ymin_a, ymin_b, part_a, part_b, gamma, beta):
    half_a = lambda b: (jnp.minimum(b, _HB - 1), 0, 0)
    half_b = lambda b: (jnp.maximum(b - _HB, 0), 0, 0)
    return pl.pallas_call(
        _final_body,
        grid=(_B,),
        in_specs=[
            pl.BlockSpec((1, _N, _O), half_a),
            pl.BlockSpec((1, _N, _O), half_b),
            pl.BlockSpec((1, _N, _O), half_a),
            pl.BlockSpec((1, _N, _O), half_b),
            pl.BlockSpec((_NW * 8, 2 * _O), lambda b: (0, 0)),
            pl.BlockSpec((_NW * 8, 2 * _O), lambda b: (0, 0)),
            pl.BlockSpec((1, _O), lambda b: (0, 0)),
            pl.BlockSpec((1, _O), lambda b: (0, 0)),
        ],
        out_specs=pl.BlockSpec((1, _O, _N), lambda b: (b, 0, 0)),
        out_shape=jax.ShapeDtypeStruct((_B, _O, _N), jnp.float32),
    )(ymax_a, ymax_b, ymin_a, ymin_b, part_a, part_b, gamma, beta)


def kernel(x, W, gamma, beta):
    bnh = _HB * _N
    idx_a, z12_a = _knn_call(x[:_HB], W)
    ymax_a, ymin_a, part_a = _sc_call(
        z12_a.reshape(bnh, 2 * _O), idx_a.reshape(bnh * _K))
    idx_b, z12_b = _knn_call(x[_HB:], W)
    ymax_b, ymin_b, part_b = _sc_call(
        z12_b.reshape(bnh, 2 * _O), idx_b.reshape(bnh * _K))
    return _final_call(
        ymax_a.reshape(_HB, _N, _O), ymax_b.reshape(_HB, _N, _O),
        ymin_a.reshape(_HB, _N, _O), ymin_b.reshape(_HB, _N, _O),
        part_a, part_b, gamma.reshape(1, _O), beta.reshape(1, _O))
